# baseline (speedup 1.0000x reference)
import functools

import jax
import jax.numpy as jnp
from jax import lax
from jax.experimental import pallas as pl
from jax.experimental.pallas import tpu as pltpu
from jax.experimental.pallas import tpu_sc as plsc

NC = 2
NS = 16
NW = NC * NS
LANES = 16
CHUNK = 64
NBUF = 6
ECHUNK = 128
EBUF = 3
DEG_COLS = 16

_SELU_ALPHA = 1.6732632423543772
_SELU_SCALE = 1.0507009873554805


def _plan(n_chunks, nbuf):
    base = (n_chunks // (NW * nbuf)) * nbuf
    extra = n_chunks - base * NW
    assert 0 <= extra <= NW
    return base, extra


def _sc_feat_deg(feat, src, dst):
    n, df = feat.shape
    n_edges = src.shape[0]
    assert n_edges % CHUNK == 0 and df % LANES == 0
    base_per_w, extra = _plan(n_edges // CHUNK, NBUF)
    outer_iters = base_per_w // NBUF
    rows_per_sub = n // NS
    wfull = rows_per_sub // CHUNK
    wrem = rows_per_sub - wfull * CHUNK

    mesh = plsc.VectorSubcoreMesh(
        core_axis_name="c", subcore_axis_name="s",
        num_cores=NC, num_subcores=NS)

    scratch = [
        pltpu.VMEM_SHARED((n, df), jnp.float32),
    ]
    for _ in range(NBUF):
        scratch += [
            pltpu.VMEM((CHUNK,), jnp.int32),
            pltpu.VMEM((CHUNK,), jnp.int32),
            pltpu.VMEM((CHUNK, df), jnp.float32),
            pltpu.SemaphoreType.DMA,
            pltpu.SemaphoreType.DMA,
            pltpu.SemaphoreType.DMA,
        ]

    @functools.partial(
        pl.kernel,
        out_type=jax.ShapeDtypeStruct((NC, n, df), jnp.float32),
        mesh=mesh,
        compiler_params=pltpu.CompilerParams(use_tc_tiling_on_sc=False),
        scratch_types=scratch,
    )
    def sc_kernel(feat_hbm, src_hbm, dst_hbm, outf_hbm,
                  acc_f, *bufs):
        src_v = [bufs[6 * b + 0] for b in range(NBUF)]
        dst_v = [bufs[6 * b + 1] for b in range(NBUF)]
        rows_v = [bufs[6 * b + 2] for b in range(NBUF)]
        sem_ld = [bufs[6 * b + 3] for b in range(NBUF)]
        sem_g = [bufs[6 * b + 4] for b in range(NBUF)]
        sem_s = [bufs[6 * b + 5] for b in range(NBUF)]

        cid = lax.axis_index("c")
        sid = lax.axis_index("s")
        wid = sid * NC + cid
        c0 = wid * base_per_w
        nck = base_per_w

        zvec = jnp.zeros((LANES,), jnp.float32)

        def fillrow(i, _):
            for j in range(df // LANES):
                rows_v[0][i, pl.ds(j * LANES, LANES)] = zvec
            return _

        lax.fori_loop(0, CHUNK, fillrow, 0)
        row0 = sid * rows_per_sub

        def zacc(k, _):
            pltpu.sync_copy(rows_v[0], acc_f.at[pl.ds(row0 + k * CHUNK,
                                                      CHUNK)])
            return _

        lax.fori_loop(0, wfull, zacc, 0)
        if wrem:
            pltpu.sync_copy(rows_v[0].at[pl.ds(0, wrem)],
                            acc_f.at[pl.ds(row0 + wfull * CHUNK, wrem)])
        plsc.subcore_barrier()

        def issue_loads(c, b):
            base = c * CHUNK
            pltpu.async_copy(src_hbm.at[pl.ds(base, CHUNK)], src_v[b],
                             sem_ld[b])
            pltpu.async_copy(dst_hbm.at[pl.ds(base, CHUNK)], dst_v[b],
                             sem_ld[b])

        def wait_loads(b):
            pltpu.make_async_copy(src_hbm.at[pl.ds(0, CHUNK)], src_v[b],
                                  sem_ld[b]).wait()
            pltpu.make_async_copy(dst_hbm.at[pl.ds(0, CHUNK)], dst_v[b],
                                  sem_ld[b]).wait()

        def issue_gather(b):
            pltpu.async_copy(feat_hbm.at[src_v[b]], rows_v[b], sem_g[b])

        def wait_gather(b):
            pltpu.make_async_copy(feat_hbm.at[src_v[b]], rows_v[b],
                                  sem_g[b]).wait()

        def issue_scatters(b):
            pltpu.async_copy(rows_v[b], acc_f.at[dst_v[b]], sem_s[b],
                             add=True)

        def wait_scatters(b):
            pltpu.make_async_copy(rows_v[b], acc_f.at[dst_v[b]],
                                  sem_s[b]).wait()

        for b in range(4):
            issue_loads(c0 + b, b)
        for b in range(3):
            wait_loads(b)
            issue_gather(b)

        def outer(o, carry):
            g0 = o * NBUF
            for k in range(NBUF):
                g = g0 + k
                b3 = (k + 3) % NBUF

                @pl.when(g + 3 < nck)
                def _():
                    wait_loads(b3)
                    issue_gather(b3)

                wait_gather(k)
                issue_scatters(k)
                b4 = (k + 4) % NBUF

                @pl.when(g + 4 < nck)
                def _():
                    @pl.when(g + 4 >= NBUF)
                    def _():
                        wait_scatters(b4)
                    issue_loads(c0 + g + 4, b4)
            return carry

        lax.fori_loop(0, outer_iters, outer, 0)

        for c in range(nck - NBUF, nck):
            wait_scatters(c % NBUF)

        if extra:
            @pl.when(wid < extra)
            def _():
                base = (NW * base_per_w + wid) * CHUNK
                pltpu.sync_copy(src_hbm.at[pl.ds(base, CHUNK)], src_v[0])
                pltpu.sync_copy(dst_hbm.at[pl.ds(base, CHUNK)], dst_v[0])
                pltpu.async_copy(feat_hbm.at[src_v[0]], rows_v[0],
                                 sem_g[0]).wait()
                pltpu.sync_copy(rows_v[0], acc_f.at[dst_v[0]], add=True)

        plsc.subcore_barrier()

        def wout_block(r, nrows):
            pltpu.sync_copy(acc_f.at[pl.ds(r, nrows)],
                            rows_v[0].at[pl.ds(0, nrows)])
            pltpu.sync_copy(rows_v[0].at[pl.ds(0, nrows)],
                            outf_hbm.at[cid, pl.ds(r, nrows)])

        def wout(k, _):
            wout_block(row0 + k * CHUNK, CHUNK)
            return _

        lax.fori_loop(0, wfull, wout, 0)
        if wrem:
            wout_block(row0 + wfull * CHUNK, wrem)

    return sc_kernel(feat, src, dst)


def _sc_edge_sum(dst, edge_attr, n, tok):
    n_edges, de = edge_attr.shape
    assert n_edges % ECHUNK == 0 and de == LANES
    base_per_w, extra = _plan(n_edges // ECHUNK, EBUF)
    outer_iters = base_per_w // EBUF
    rows_per_sub = n // NS
    wfull = rows_per_sub // ECHUNK
    wrem = rows_per_sub - wfull * ECHUNK

    mesh = plsc.VectorSubcoreMesh(
        core_axis_name="c", subcore_axis_name="s",
        num_cores=NC, num_subcores=NS)

    scratch = [
        pltpu.VMEM_SHARED((n, de), jnp.float32),
        pltpu.VMEM_SHARED((n, DEG_COLS), jnp.float32),
        pltpu.VMEM((ECHUNK, DEG_COLS), jnp.float32),
        pltpu.VMEM((ECHUNK, DEG_COLS), jnp.float32),
    ]
    for _ in range(EBUF):
        scratch += [
            pltpu.VMEM((ECHUNK,), jnp.int32),
            pltpu.VMEM((ECHUNK, de), jnp.float32),
            pltpu.SemaphoreType.DMA,
            pltpu.SemaphoreType.DMA,
        ]

    @functools.partial(
        pl.kernel,
        out_type=(
            jax.ShapeDtypeStruct((NC, n, de), jnp.float32),
            jax.ShapeDtypeStruct((NC, n, DEG_COLS), jnp.float32),
        ),
        mesh=mesh,
        compiler_params=pltpu.CompilerParams(use_tc_tiling_on_sc=False),
        scratch_types=scratch,
    )
    def sc_kernel(dst_hbm, ea_hbm, tok_hbm, oute_hbm, outd_hbm,
                  acc_e, acc_d, ones_v, zed_v, *bufs):
        dst_v = [bufs[4 * b + 0] for b in range(EBUF)]
        edge_v = [bufs[4 * b + 1] for b in range(EBUF)]
        sem_ld = [bufs[4 * b + 2] for b in range(EBUF)]
        sem_s = [bufs[4 * b + 3] for b in range(EBUF)]

        cid = lax.axis_index("c")
        sid = lax.axis_index("s")
        wid = sid * NC + cid
        c0 = wid * base_per_w
        nck = base_per_w

        zvec = jnp.zeros((LANES,), jnp.float32)

        def fillrow(i, _):
            edge_v[0][i, :] = zvec
            ones_v[i, :] = jnp.ones((LANES,), jnp.float32)
            zed_v[i, :] = zvec
            return _

        lax.fori_loop(0, ECHUNK, fillrow, 0)
        row0 = sid * rows_per_sub

        def zacc(k, _):
            r = row0 + k * ECHUNK
            pltpu.sync_copy(edge_v[0], acc_e.at[pl.ds(r, ECHUNK)])
            pltpu.sync_copy(zed_v, acc_d.at[pl.ds(r, ECHUNK)])
            return _

        lax.fori_loop(0, wfull, zacc, 0)
        if wrem:
            r = row0 + wfull * ECHUNK
            pltpu.sync_copy(edge_v[0].at[pl.ds(0, wrem)],
                            acc_e.at[pl.ds(r, wrem)])
            pltpu.sync_copy(zed_v.at[pl.ds(0, wrem)],
                            acc_d.at[pl.ds(r, wrem)])
        plsc.subcore_barrier()

        def issue_loads(c, b):
            base = c * ECHUNK
            pltpu.async_copy(dst_hbm.at[pl.ds(base, ECHUNK)], dst_v[b],
                             sem_ld[b])
            pltpu.async_copy(ea_hbm.at[pl.ds(base, ECHUNK)], edge_v[b],
                             sem_ld[b])

        def wait_loads(b):
            pltpu.make_async_copy(dst_hbm.at[pl.ds(0, ECHUNK)], dst_v[b],
                                  sem_ld[b]).wait()
            pltpu.make_async_copy(ea_hbm.at[pl.ds(0, ECHUNK)], edge_v[b],
                                  sem_ld[b]).wait()

        def issue_scatter(b):
            pltpu.async_copy(edge_v[b], acc_e.at[dst_v[b]], sem_s[b],
                             add=True)
            pltpu.async_copy(ones_v, acc_d.at[dst_v[b]], sem_s[b],
                             add=True)

        def wait_scatter(b):
            pltpu.make_async_copy(edge_v[b], acc_e.at[dst_v[b]],
                                  sem_s[b]).wait()
            pltpu.make_async_copy(ones_v, acc_d.at[dst_v[b]],
                                  sem_s[b]).wait()

        issue_loads(c0, 0)
        issue_loads(c0 + 1, 1)

        def outer(o, carry):
            g0 = o * EBUF
            for k in range(EBUF):
                g = g0 + k
                wait_loads(k)
                issue_scatter(k)
                b2 = (k + 2) % EBUF

                @pl.when(g + 2 < nck)
                def _():
                    @pl.when(g + 2 >= EBUF)
                    def _():
                        wait_scatter(b2)
                    issue_loads(c0 + g + 2, b2)
            return carry

        lax.fori_loop(0, outer_iters, outer, 0)

        for c in range(nck - EBUF, nck):
            wait_scatter(c % EBUF)

        if extra:
            @pl.when(wid < extra)
            def _():
                base = (NW * base_per_w + wid) * ECHUNK
                pltpu.sync_copy(dst_hbm.at[pl.ds(base, ECHUNK)], dst_v[0])
                pltpu.sync_copy(ea_hbm.at[pl.ds(base, ECHUNK)], edge_v[0])
                pltpu.sync_copy(edge_v[0], acc_e.at[dst_v[0]], add=True)
                pltpu.sync_copy(ones_v, acc_d.at[dst_v[0]], add=True)

        plsc.subcore_barrier()

        def wout_block(r, nrows):
            pltpu.sync_copy(acc_e.at[pl.ds(r, nrows)],
                            edge_v[0].at[pl.ds(0, nrows)])
            pltpu.sync_copy(edge_v[0].at[pl.ds(0, nrows)],
                            oute_hbm.at[cid, pl.ds(r, nrows)])
            pltpu.sync_copy(acc_d.at[pl.ds(r, nrows)],
                            zed_v.at[pl.ds(0, nrows)])
            pltpu.sync_copy(zed_v.at[pl.ds(0, nrows)],
                            outd_hbm.at[cid, pl.ds(r, nrows)])

        def wout(k, _):
            wout_block(row0 + k * ECHUNK, ECHUNK)
            return _

        lax.fori_loop(0, wfull, wout, 0)
        if wrem:
            wout_block(row0 + wfull * ECHUNK, wrem)

    return sc_kernel(dst, edge_attr, tok)


def _tc_matmul_feat(part_f, weight):
    n = part_f.shape[1]
    df = part_f.shape[2]
    d_out = weight.shape[1]
    de = weight.shape[0] - df
    block_rows = 2000
    assert n % block_rows == 0

    def tc_body(pf_ref, w_ref, o_ref):
        s = pf_ref[0] + pf_ref[1]
        o_ref[...] = jnp.dot(s, w_ref[de:, :],
                             preferred_element_type=jnp.float32,
                             precision=lax.Precision.HIGHEST)

    return pl.pallas_call(
        tc_body,
        grid=(n // block_rows,),
        in_specs=[
            pl.BlockSpec((NC, block_rows, df), lambda i: (0, i, 0)),
            pl.BlockSpec((de + df, d_out), lambda i: (0, 0)),
        ],
        out_specs=pl.BlockSpec((block_rows, d_out), lambda i: (i, 0)),
        out_shape=jax.ShapeDtypeStruct((n, d_out), jnp.float32),
    )(part_f, weight)


def _tc_finish(h_feat, part_e, part_d, weight, bias):
    n, d_out = h_feat.shape
    de = part_e.shape[2]
    block_rows = 2000
    assert n % block_rows == 0

    def tc_body(hf_ref, pe_ref, pd_ref, w_ref, b_ref, o_ref):
        se = pe_ref[0] + pe_ref[1]
        deg = pd_ref[0, :, 0:1] + pd_ref[1, :, 0:1]
        norm2 = 1.0 / jnp.maximum(deg, 1.0)
        h = hf_ref[...] + jnp.dot(se, w_ref[:de, :],
                                  preferred_element_type=jnp.float32,
                                  precision=lax.Precision.HIGHEST)
        h = h * norm2 + b_ref[...]
        o_ref[...] = _SELU_SCALE * jnp.where(
            h > 0.0, h, _SELU_ALPHA * (jnp.exp(h) - 1.0))

    return pl.pallas_call(
        tc_body,
        grid=(n // block_rows,),
        in_specs=[
            pl.BlockSpec((block_rows, d_out), lambda i: (i, 0)),
            pl.BlockSpec((NC, block_rows, de), lambda i: (0, i, 0)),
            pl.BlockSpec((NC, block_rows, DEG_COLS), lambda i: (0, i, 0)),
            pl.BlockSpec((de + d_out, d_out), lambda i: (0, 0)),
            pl.BlockSpec((1, d_out), lambda i: (0, 0)),
        ],
        out_specs=pl.BlockSpec((block_rows, d_out), lambda i: (i, 0)),
        out_shape=jax.ShapeDtypeStruct((n, d_out), jnp.float32),
    )(h_feat, part_e, part_d, weight, bias.reshape(1, d_out))


def kernel(feat, edge_index, edge_attr, weight, bias):
    src = edge_index[0]
    dst = edge_index[1]
    part_f = _sc_feat_deg(feat, src, dst)
    tok = lax.slice(part_f, (0, 0, 0), (1, 8, part_f.shape[2]))
    part_e, part_d = _sc_edge_sum(dst, edge_attr, feat.shape[0], tok)
    h_feat = _tc_matmul_feat(part_f, weight)
    return _tc_finish(h_feat, part_e, part_d, weight, bias)


# speedup vs baseline: 1.0185x; 1.0185x over previous
import functools

import jax
import jax.numpy as jnp
from jax import lax
from jax.experimental import pallas as pl
from jax.experimental.pallas import tpu as pltpu
from jax.experimental.pallas import tpu_sc as plsc

NC = 2
NS = 16
NW = NC * NS
LANES = 16
CHUNK = 80
NBUF = 4
ECHUNK = 128
EBUF = 3
DEG_COLS = 16

_SELU_ALPHA = 1.6732632423543772
_SELU_SCALE = 1.0507009873554805


def _plan(n_chunks, nbuf):
    base = (n_chunks // (NW * nbuf)) * nbuf
    extra = n_chunks - base * NW
    assert 0 <= extra <= NW
    return base, extra


def _sc_feat_deg(feat, ei):
    n, df = feat.shape
    n_edges = ei.shape[1]
    assert n_edges % CHUNK == 0 and df % LANES == 0
    base_per_w, extra = _plan(n_edges // CHUNK, NBUF)
    outer_iters = base_per_w // NBUF
    rows_per_sub = n // NS
    wfull = rows_per_sub // CHUNK
    wrem = rows_per_sub - wfull * CHUNK

    mesh = plsc.VectorSubcoreMesh(
        core_axis_name="c", subcore_axis_name="s",
        num_cores=NC, num_subcores=NS)

    scratch = [
        pltpu.VMEM_SHARED((n, df), jnp.float32),
    ]
    for _ in range(NBUF):
        scratch += [
            pltpu.VMEM((CHUNK,), jnp.int32),
            pltpu.VMEM((CHUNK,), jnp.int32),
            pltpu.VMEM((CHUNK, df), jnp.float32),
            pltpu.SemaphoreType.DMA,
            pltpu.SemaphoreType.DMA,
            pltpu.SemaphoreType.DMA,
        ]

    @functools.partial(
        pl.kernel,
        out_type=jax.ShapeDtypeStruct((NC, n, df), jnp.float32),
        mesh=mesh,
        compiler_params=pltpu.CompilerParams(use_tc_tiling_on_sc=False),
        scratch_types=scratch,
    )
    def sc_kernel(feat_hbm, ei_hbm, outf_hbm,
                  acc_f, *bufs):
        src_v = [bufs[6 * b + 0] for b in range(NBUF)]
        dst_v = [bufs[6 * b + 1] for b in range(NBUF)]
        rows_v = [bufs[6 * b + 2] for b in range(NBUF)]
        sem_ld = [bufs[6 * b + 3] for b in range(NBUF)]
        sem_g = [bufs[6 * b + 4] for b in range(NBUF)]
        sem_s = [bufs[6 * b + 5] for b in range(NBUF)]

        cid = lax.axis_index("c")
        sid = lax.axis_index("s")
        wid = sid * NC + cid
        c0 = wid * base_per_w
        nck = base_per_w

        zvec = jnp.zeros((LANES,), jnp.float32)

        def fillrow(i, _):
            for j in range(df // LANES):
                rows_v[0][i, pl.ds(j * LANES, LANES)] = zvec
            return _

        lax.fori_loop(0, CHUNK, fillrow, 0)
        row0 = sid * rows_per_sub

        def zacc(k, _):
            pltpu.sync_copy(rows_v[0], acc_f.at[pl.ds(row0 + k * CHUNK,
                                                      CHUNK)])
            return _

        lax.fori_loop(0, wfull, zacc, 0)
        if wrem:
            pltpu.sync_copy(rows_v[0].at[pl.ds(0, wrem)],
                            acc_f.at[pl.ds(row0 + wfull * CHUNK, wrem)])
        plsc.subcore_barrier()

        def issue_loads(c, b):
            base = c * CHUNK
            pltpu.async_copy(ei_hbm.at[0, pl.ds(base, CHUNK)], src_v[b],
                             sem_ld[b])
            pltpu.async_copy(ei_hbm.at[1, pl.ds(base, CHUNK)], dst_v[b],
                             sem_ld[b])

        def wait_loads(b):
            pltpu.make_async_copy(ei_hbm.at[0, pl.ds(0, CHUNK)], src_v[b],
                                  sem_ld[b]).wait()
            pltpu.make_async_copy(ei_hbm.at[1, pl.ds(0, CHUNK)], dst_v[b],
                                  sem_ld[b]).wait()

        def issue_gather(b):
            pltpu.async_copy(feat_hbm.at[src_v[b]], rows_v[b], sem_g[b])

        def wait_gather(b):
            pltpu.make_async_copy(feat_hbm.at[src_v[b]], rows_v[b],
                                  sem_g[b]).wait()

        def issue_scatters(b):
            pltpu.async_copy(rows_v[b], acc_f.at[dst_v[b]], sem_s[b],
                             add=True)

        def wait_scatters(b):
            pltpu.make_async_copy(rows_v[b], acc_f.at[dst_v[b]],
                                  sem_s[b]).wait()

        issue_loads(c0, 0)
        issue_loads(c0 + 1, 1)
        issue_loads(c0 + 2, 2)
        wait_loads(0)
        issue_gather(0)
        wait_loads(1)
        issue_gather(1)

        def outer(o, carry):
            g0 = o * NBUF
            for k in range(NBUF):
                g = g0 + k
                b2 = (k + 2) % NBUF

                @pl.when(g + 2 < nck)
                def _():
                    wait_loads(b2)
                    issue_gather(b2)

                wait_gather(k)
                issue_scatters(k)
                b3 = (k + 3) % NBUF

                @pl.when(g + 3 < nck)
                def _():
                    @pl.when(g + 3 >= NBUF)
                    def _():
                        wait_scatters(b3)
                    issue_loads(c0 + g + 3, b3)
            return carry

        lax.fori_loop(0, outer_iters, outer, 0)

        for c in range(nck - NBUF, nck):
            wait_scatters(c % NBUF)

        if extra:
            @pl.when(wid < extra)
            def _():
                base = (NW * base_per_w + wid) * CHUNK
                pltpu.sync_copy(ei_hbm.at[0, pl.ds(base, CHUNK)], src_v[0])
                pltpu.sync_copy(ei_hbm.at[1, pl.ds(base, CHUNK)], dst_v[0])
                pltpu.async_copy(feat_hbm.at[src_v[0]], rows_v[0],
                                 sem_g[0]).wait()
                pltpu.sync_copy(rows_v[0], acc_f.at[dst_v[0]], add=True)

        plsc.subcore_barrier()

        def wout_block(r, nrows):
            pltpu.sync_copy(acc_f.at[pl.ds(r, nrows)],
                            rows_v[0].at[pl.ds(0, nrows)])
            pltpu.sync_copy(rows_v[0].at[pl.ds(0, nrows)],
                            outf_hbm.at[cid, pl.ds(r, nrows)])

        def wout(k, _):
            wout_block(row0 + k * CHUNK, CHUNK)
            return _

        lax.fori_loop(0, wfull, wout, 0)
        if wrem:
            wout_block(row0 + wfull * CHUNK, wrem)

    return sc_kernel(feat, ei)


def _sc_edge_sum(ei, edge_attr, n, tok):
    n_edges, de = edge_attr.shape
    assert n_edges % ECHUNK == 0 and de == LANES
    base_per_w, extra = _plan(n_edges // ECHUNK, EBUF)
    outer_iters = base_per_w // EBUF
    rows_per_sub = n // NS
    wfull = rows_per_sub // ECHUNK
    wrem = rows_per_sub - wfull * ECHUNK

    mesh = plsc.VectorSubcoreMesh(
        core_axis_name="c", subcore_axis_name="s",
        num_cores=NC, num_subcores=NS)

    scratch = [
        pltpu.VMEM_SHARED((n, de), jnp.float32),
        pltpu.VMEM_SHARED((n, DEG_COLS), jnp.float32),
        pltpu.VMEM((ECHUNK, DEG_COLS), jnp.float32),
        pltpu.VMEM((ECHUNK, DEG_COLS), jnp.float32),
    ]
    for _ in range(EBUF):
        scratch += [
            pltpu.VMEM((ECHUNK,), jnp.int32),
            pltpu.VMEM((ECHUNK, de), jnp.float32),
            pltpu.SemaphoreType.DMA,
            pltpu.SemaphoreType.DMA,
        ]

    @functools.partial(
        pl.kernel,
        out_type=(
            jax.ShapeDtypeStruct((NC, n, de), jnp.float32),
            jax.ShapeDtypeStruct((NC, n, DEG_COLS), jnp.float32),
        ),
        mesh=mesh,
        compiler_params=pltpu.CompilerParams(use_tc_tiling_on_sc=False),
        scratch_types=scratch,
    )
    def sc_kernel(ei_hbm, ea_hbm, tok_hbm, oute_hbm, outd_hbm,
                  acc_e, acc_d, ones_v, zed_v, *bufs):
        dst_v = [bufs[4 * b + 0] for b in range(EBUF)]
        edge_v = [bufs[4 * b + 1] for b in range(EBUF)]
        sem_ld = [bufs[4 * b + 2] for b in range(EBUF)]
        sem_s = [bufs[4 * b + 3] for b in range(EBUF)]

        cid = lax.axis_index("c")
        sid = lax.axis_index("s")
        wid = sid * NC + cid
        c0 = wid * base_per_w
        nck = base_per_w

        zvec = jnp.zeros((LANES,), jnp.float32)

        def fillrow(i, _):
            edge_v[0][i, :] = zvec
            ones_v[i, :] = jnp.ones((LANES,), jnp.float32)
            zed_v[i, :] = zvec
            return _

        lax.fori_loop(0, ECHUNK, fillrow, 0)
        row0 = sid * rows_per_sub

        def zacc(k, _):
            r = row0 + k * ECHUNK
            pltpu.sync_copy(edge_v[0], acc_e.at[pl.ds(r, ECHUNK)])
            pltpu.sync_copy(zed_v, acc_d.at[pl.ds(r, ECHUNK)])
            return _

        lax.fori_loop(0, wfull, zacc, 0)
        if wrem:
            r = row0 + wfull * ECHUNK
            pltpu.sync_copy(edge_v[0].at[pl.ds(0, wrem)],
                            acc_e.at[pl.ds(r, wrem)])
            pltpu.sync_copy(zed_v.at[pl.ds(0, wrem)],
                            acc_d.at[pl.ds(r, wrem)])
        plsc.subcore_barrier()

        def issue_loads(c, b):
            base = c * ECHUNK
            pltpu.async_copy(ei_hbm.at[1, pl.ds(base, ECHUNK)], dst_v[b],
                             sem_ld[b])
            pltpu.async_copy(ea_hbm.at[pl.ds(base, ECHUNK)], edge_v[b],
                             sem_ld[b])

        def wait_loads(b):
            pltpu.make_async_copy(ei_hbm.at[1, pl.ds(0, ECHUNK)], dst_v[b],
                                  sem_ld[b]).wait()
            pltpu.make_async_copy(ea_hbm.at[pl.ds(0, ECHUNK)], edge_v[b],
                                  sem_ld[b]).wait()

        def issue_scatter(b):
            pltpu.async_copy(edge_v[b], acc_e.at[dst_v[b]], sem_s[b],
                             add=True)
            pltpu.async_copy(ones_v, acc_d.at[dst_v[b]], sem_s[b],
                             add=True)

        def wait_scatter(b):
            pltpu.make_async_copy(edge_v[b], acc_e.at[dst_v[b]],
                                  sem_s[b]).wait()
            pltpu.make_async_copy(ones_v, acc_d.at[dst_v[b]],
                                  sem_s[b]).wait()

        issue_loads(c0, 0)
        issue_loads(c0 + 1, 1)

        def outer(o, carry):
            g0 = o * EBUF
            for k in range(EBUF):
                g = g0 + k
                wait_loads(k)
                issue_scatter(k)
                b2 = (k + 2) % EBUF

                @pl.when(g + 2 < nck)
                def _():
                    @pl.when(g + 2 >= EBUF)
                    def _():
                        wait_scatter(b2)
                    issue_loads(c0 + g + 2, b2)
            return carry

        lax.fori_loop(0, outer_iters, outer, 0)

        for c in range(nck - EBUF, nck):
            wait_scatter(c % EBUF)

        if extra:
            @pl.when(wid < extra)
            def _():
                base = (NW * base_per_w + wid) * ECHUNK
                pltpu.sync_copy(ei_hbm.at[1, pl.ds(base, ECHUNK)], dst_v[0])
                pltpu.sync_copy(ea_hbm.at[pl.ds(base, ECHUNK)], edge_v[0])
                pltpu.sync_copy(edge_v[0], acc_e.at[dst_v[0]], add=True)
                pltpu.sync_copy(ones_v, acc_d.at[dst_v[0]], add=True)

        plsc.subcore_barrier()

        def wout_block(r, nrows):
            pltpu.sync_copy(acc_e.at[pl.ds(r, nrows)],
                            edge_v[0].at[pl.ds(0, nrows)])
            pltpu.sync_copy(edge_v[0].at[pl.ds(0, nrows)],
                            oute_hbm.at[cid, pl.ds(r, nrows)])
            pltpu.sync_copy(acc_d.at[pl.ds(r, nrows)],
                            zed_v.at[pl.ds(0, nrows)])
            pltpu.sync_copy(zed_v.at[pl.ds(0, nrows)],
                            outd_hbm.at[cid, pl.ds(r, nrows)])

        def wout(k, _):
            wout_block(row0 + k * ECHUNK, ECHUNK)
            return _

        lax.fori_loop(0, wfull, wout, 0)
        if wrem:
            wout_block(row0 + wfull * ECHUNK, wrem)

    return sc_kernel(ei, edge_attr, tok)


def _tc_matmul_feat(part_f, weight):
    n = part_f.shape[1]
    df = part_f.shape[2]
    d_out = weight.shape[1]
    de = weight.shape[0] - df
    block_rows = 2000
    assert n % block_rows == 0

    def tc_body(pf_ref, w_ref, o_ref):
        s = pf_ref[0] + pf_ref[1]
        o_ref[...] = jnp.dot(s, w_ref[de:, :],
                             preferred_element_type=jnp.float32,
                             precision=lax.Precision.HIGHEST)

    return pl.pallas_call(
        tc_body,
        grid=(n // block_rows,),
        in_specs=[
            pl.BlockSpec((NC, block_rows, df), lambda i: (0, i, 0)),
            pl.BlockSpec((de + df, d_out), lambda i: (0, 0)),
        ],
        out_specs=pl.BlockSpec((block_rows, d_out), lambda i: (i, 0)),
        out_shape=jax.ShapeDtypeStruct((n, d_out), jnp.float32),
    )(part_f, weight)


def _tc_finish(h_feat, part_e, part_d, weight, bias):
    n, d_out = h_feat.shape
    de = part_e.shape[2]
    block_rows = 2000
    assert n % block_rows == 0

    def tc_body(hf_ref, pe_ref, pd_ref, w_ref, b_ref, o_ref):
        se = pe_ref[0] + pe_ref[1]
        deg = pd_ref[0, :, 0:1] + pd_ref[1, :, 0:1]
        norm2 = 1.0 / jnp.maximum(deg, 1.0)
        h = hf_ref[...] + jnp.dot(se, w_ref[:de, :],
                                  preferred_element_type=jnp.float32,
                                  precision=lax.Precision.HIGHEST)
        h = h * norm2 + b_ref[...]
        o_ref[...] = _SELU_SCALE * jnp.where(
            h > 0.0, h, _SELU_ALPHA * (jnp.exp(h) - 1.0))

    return pl.pallas_call(
        tc_body,
        grid=(n // block_rows,),
        in_specs=[
            pl.BlockSpec((block_rows, d_out), lambda i: (i, 0)),
            pl.BlockSpec((NC, block_rows, de), lambda i: (0, i, 0)),
            pl.BlockSpec((NC, block_rows, DEG_COLS), lambda i: (0, i, 0)),
            pl.BlockSpec((de + d_out, d_out), lambda i: (0, 0)),
            pl.BlockSpec((1, d_out), lambda i: (0, 0)),
        ],
        out_specs=pl.BlockSpec((block_rows, d_out), lambda i: (i, 0)),
        out_shape=jax.ShapeDtypeStruct((n, d_out), jnp.float32),
    )(h_feat, part_e, part_d, weight, bias.reshape(1, d_out))


def kernel(feat, edge_index, edge_attr, weight, bias):
    part_f = _sc_feat_deg(feat, edge_index)
    tok = lax.slice(part_f, (0, 0, 0), (1, 8, part_f.shape[2]))
    part_e, part_d = _sc_edge_sum(edge_index, edge_attr, feat.shape[0], tok)
    h_feat = _tc_matmul_feat(part_f, weight)
    return _tc_finish(h_feat, part_e, part_d, weight, bias)
